# baseline (speedup 1.0000x reference)
import functools

import jax
import jax.numpy as jnp
from jax import lax
from jax.experimental import pallas as pl
from jax.experimental.pallas import tpu as pltpu
from jax.experimental.pallas import tpu_sc as plsc

N = 10000
N_PAD = 10112
E = 320000
D = 128
NC = 2
NS = 16
NW = NC * NS
CHUNK = 128
RPT = 80
E_PAD = NW * RPT * CHUNK
SLAB = N_PAD // NS


def _sc_kernel(x_hbm, src_hbm, dst_hbm, w_hbm, part_hbm,
               src_v, dst_v, w_v, rows_v, acc, gsem):
    c = lax.axis_index("c")
    s = lax.axis_index("s")
    wid = c * NS + s
    base = wid * RPT

    pltpu.sync_copy(src_hbm.at[pl.ds(base * CHUNK, RPT * CHUNK)], src_v)
    pltpu.sync_copy(dst_hbm.at[pl.ds(base, RPT)], dst_v)
    pltpu.sync_copy(w_hbm.at[pl.ds(base * CHUNK, RPT * CHUNK)], w_v)

    def zrow(r, _):
        for cc in range(8):
            rows_v[r, pl.ds(cc * 16, 16)] = jnp.zeros((16,), jnp.float32)
        return 0
    lax.fori_loop(0, CHUNK, zrow, 0)
    for t in range(4):
        pltpu.sync_copy(rows_v, acc.at[pl.ds(s * SLAB + t * CHUNK, CHUNK)])
    pltpu.sync_copy(rows_v.at[pl.ds(0, SLAB - 4 * CHUNK)],
                    acc.at[pl.ds(s * SLAB + 4 * CHUNK, SLAB - 4 * CHUNK)])
    plsc.subcore_barrier()

    def body(j, _):
        pltpu.async_copy(
            x_hbm.at[src_v.at[pl.ds(j * CHUNK, CHUNK)]], rows_v, gsem
        ).wait()

        def mul_body(g, _):
            w16 = w_v[pl.ds(j * CHUNK + g * 16, 16)]
            for kk in range(16):
                wv = jnp.broadcast_to(w16[kk], (16,))
                row = g * 16 + kk
                for cc in range(8):
                    sl = pl.ds(cc * 16, 16)
                    rows_v[row, sl] = rows_v[row, sl] * wv
            return 0
        lax.fori_loop(0, CHUNK // 16, mul_body, 0)

        pltpu.sync_copy(rows_v, acc.at[dst_v.at[j]], add=True)
        return 0

    lax.fori_loop(0, RPT, body, 0)
    plsc.subcore_barrier()

    pltpu.sync_copy(acc.at[pl.ds(s * SLAB, SLAB)],
                    part_hbm.at[c, pl.ds(s * SLAB, SLAB)])


def _combine_body(p_ref, o_ref):
    o_ref[...] = p_ref[0] + p_ref[1]


def kernel(edge_index, x, edge_weight):
    pad = E_PAD - E
    pad_idx = (jnp.arange(pad, dtype=jnp.int32) % N)
    src = jnp.concatenate([edge_index[0].astype(jnp.int32), pad_idx])
    dst = jnp.concatenate([edge_index[1].astype(jnp.int32), pad_idx])
    w = jnp.concatenate([edge_weight, jnp.zeros((pad,), jnp.float32)])
    rtot = E_PAD // CHUNK
    dst2 = dst.reshape(rtot, CHUNK)

    mesh = plsc.VectorSubcoreMesh(core_axis_name="c", subcore_axis_name="s",
                                  num_cores=NC, num_subcores=NS)
    part = pl.kernel(
        _sc_kernel,
        out_type=jax.ShapeDtypeStruct((NC, N_PAD, D), jnp.float32),
        mesh=mesh,
        scratch_types=[
            pltpu.VMEM((RPT * CHUNK,), jnp.int32),
            pltpu.VMEM((RPT, CHUNK), jnp.int32),
            pltpu.VMEM((RPT * CHUNK,), jnp.float32),
            pltpu.VMEM((CHUNK, D), jnp.float32),
            pltpu.VMEM_SHARED((N_PAD, D), jnp.float32),
            pltpu.SemaphoreType.DMA,
        ],
    )(x, src, dst2, w)

    out = pl.pallas_call(
        _combine_body,
        grid=(10,),
        in_specs=[pl.BlockSpec((NC, N // 10, D), lambda i: (0, i, 0))],
        out_specs=pl.BlockSpec((N // 10, D), lambda i: (i, 0)),
        out_shape=jax.ShapeDtypeStruct((N, D), jnp.float32),
    )(part)
    return out


# speedup vs baseline: 1.3470x; 1.3470x over previous
import functools

import jax
import jax.numpy as jnp
from jax import lax
from jax.experimental import pallas as pl
from jax.experimental.pallas import tpu as pltpu
from jax.experimental.pallas import tpu_sc as plsc

N = 10000
N_PAD = 10112
E = 320000
D = 128
NC = 2
NS = 16
NW = NC * NS
CHUNK = 128
RPT = 80
E_PAD = NW * RPT * CHUNK
SLAB = N_PAD // NS
NBUF = 2
MDEPTH = 4


def _sc_kernel(x_hbm, src_hbm, dst_hbm, w_hbm, part_hbm,
               src_r, dst_v, w_r, rows0, rows1, acc,
               g0, g1, s0, s1, ms0, ms1, ms2, ms3, mw0, mw1, mw2, mw3):
    c = lax.axis_index("c")
    s = lax.axis_index("s")
    wid = c * NS + s
    base = wid * RPT
    rows = (rows0, rows1)
    gsem = (g0, g1)
    ssem = (s0, s1)
    msem = (ms0, ms1, ms2, ms3)
    wsem = (mw0, mw1, mw2, mw3)

    pltpu.sync_copy(dst_hbm.at[pl.ds(base, RPT)], dst_v)
    pltpu.sync_copy(src_hbm.at[pl.ds(base * CHUNK, CHUNK)],
                    src_r.at[pl.ds(0, CHUNK)])
    pltpu.sync_copy(w_hbm.at[pl.ds(base * CHUNK, CHUNK)],
                    w_r.at[pl.ds(0, CHUNK)])

    def zrow(r, _):
        for cc in range(8):
            rows0[r, pl.ds(cc * 16, 16)] = jnp.zeros((16,), jnp.float32)
        return 0
    lax.fori_loop(0, CHUNK, zrow, 0)
    for t in range(4):
        pltpu.sync_copy(rows0, acc.at[pl.ds(s * SLAB + t * CHUNK, CHUNK)])
    pltpu.sync_copy(rows0.at[pl.ds(0, SLAB - 4 * CHUNK)],
                    acc.at[pl.ds(s * SLAB + 4 * CHUNK, SLAB - 4 * CHUNK)])

    def g_start(j, b, m):
        return pltpu.async_copy(
            x_hbm.at[src_r.at[pl.ds(m * CHUNK, CHUNK)]], rows[b], gsem[b])

    def g_wait(j, b, m):
        pltpu.make_async_copy(
            x_hbm.at[src_r.at[pl.ds(m * CHUNK, CHUNK)]], rows[b],
            gsem[b]).wait()

    def s_start(j, b):
        return pltpu.async_copy(rows[b], acc.at[dst_v.at[j]], ssem[b],
                                add=True)

    def s_wait(j, b):
        pltpu.make_async_copy(rows[b], acc.at[dst_v.at[j]], ssem[b]).wait()

    def m_start(j, m):
        pltpu.async_copy(src_hbm.at[pl.ds((base + j) * CHUNK, CHUNK)],
                         src_r.at[pl.ds(m * CHUNK, CHUNK)], msem[m])
        pltpu.async_copy(w_hbm.at[pl.ds((base + j) * CHUNK, CHUNK)],
                         w_r.at[pl.ds(m * CHUNK, CHUNK)], wsem[m])

    def m_wait(j, m):
        pltpu.make_async_copy(src_hbm.at[pl.ds((base + j) * CHUNK, CHUNK)],
                              src_r.at[pl.ds(m * CHUNK, CHUNK)],
                              msem[m]).wait()
        pltpu.make_async_copy(w_hbm.at[pl.ds((base + j) * CHUNK, CHUNK)],
                              w_r.at[pl.ds(m * CHUNK, CHUNK)],
                              wsem[m]).wait()

    g_start(0, 0, 0)
    for m in range(1, MDEPTH):
        m_start(m, m)
    plsc.subcore_barrier()

    def mul_half(b, u, lo, hi):
        def mul_body(g, _, b=b, u=u):
            w16 = w_r[pl.ds(u * CHUNK + g * 16, 16)]
            for kk in range(16):
                wv = jnp.broadcast_to(w16[kk], (16,))
                row = g * 16 + kk
                for cc in range(8):
                    sl = pl.ds(cc * 16, 16)
                    rows[b][row, sl] = rows[b][row, sl] * wv
            return 0
        lax.fori_loop(lo, hi, mul_body, 0)

    def body(jj, _):
        for u in range(4):
            j = jj * 4 + u
            b = u % 2
            nb = (u + 1) % 2
            mn = (u + 1) % 4

            g_wait(j, b, u)
            mul_half(b, u, 0, 4)

            if u == 0:
                @pl.when(jj > 0)
                def _(j=j, nb=nb, mn=mn):
                    s_wait(j - 1, nb)
                    m_wait(j + 1, mn)
                    g_start(j + 1, nb, mn)

                @pl.when(jj == 0)
                def _(nb=nb, mn=mn):
                    m_wait(1, mn)
                    g_start(1, nb, mn)
            elif u == 3:
                @pl.when(jj < RPT // 4 - 1)
                def _(j=j, nb=nb, mn=mn):
                    s_wait(j - 1, nb)
                    m_wait(j + 1, mn)
                    g_start(j + 1, nb, mn)
            else:
                s_wait(j - 1, nb)
                m_wait(j + 1, mn)
                g_start(j + 1, nb, mn)

            mul_half(b, u, 4, 8)
            s_start(j, b)

            @pl.when(jj < RPT // 4 - 1)
            def _(j=j, u=u):
                m_start(j + 4, u)
        return 0

    lax.fori_loop(0, RPT // 4, body, 0)

    s_wait(RPT - 2, 0)
    s_wait(RPT - 1, 1)
    plsc.subcore_barrier()

    pltpu.sync_copy(acc.at[pl.ds(s * SLAB, SLAB)],
                    part_hbm.at[c, pl.ds(s * SLAB, SLAB)])


def _combine_body(p_ref, o_ref):
    o_ref[...] = p_ref[0] + p_ref[1]


def kernel(edge_index, x, edge_weight):
    pad = E_PAD - E
    pad_idx = (jnp.arange(pad, dtype=jnp.int32) % N)
    src = jnp.concatenate([edge_index[0].astype(jnp.int32), pad_idx])
    dst = jnp.concatenate([edge_index[1].astype(jnp.int32), pad_idx])
    w = jnp.concatenate([edge_weight, jnp.zeros((pad,), jnp.float32)])
    rtot = E_PAD // CHUNK
    dst2 = dst.reshape(rtot, CHUNK)

    mesh = plsc.VectorSubcoreMesh(core_axis_name="c", subcore_axis_name="s",
                                  num_cores=NC, num_subcores=NS)
    part = pl.kernel(
        _sc_kernel,
        out_type=jax.ShapeDtypeStruct((NC, N_PAD, D), jnp.float32),
        mesh=mesh,
        scratch_types=[
            pltpu.VMEM((MDEPTH * CHUNK,), jnp.int32),
            pltpu.VMEM((RPT, CHUNK), jnp.int32),
            pltpu.VMEM((MDEPTH * CHUNK,), jnp.float32),
            pltpu.VMEM((CHUNK, D), jnp.float32),
            pltpu.VMEM((CHUNK, D), jnp.float32),
            pltpu.VMEM_SHARED((N_PAD, D), jnp.float32),
            pltpu.SemaphoreType.DMA,
            pltpu.SemaphoreType.DMA,
            pltpu.SemaphoreType.DMA,
            pltpu.SemaphoreType.DMA,
            pltpu.SemaphoreType.DMA,
            pltpu.SemaphoreType.DMA,
            pltpu.SemaphoreType.DMA,
            pltpu.SemaphoreType.DMA,
            pltpu.SemaphoreType.DMA,
            pltpu.SemaphoreType.DMA,
            pltpu.SemaphoreType.DMA,
            pltpu.SemaphoreType.DMA,
        ],
    )(x, src, dst2, w)

    out = pl.pallas_call(
        _combine_body,
        grid=(10,),
        in_specs=[pl.BlockSpec((NC, N // 10, D), lambda i: (0, i, 0))],
        out_specs=pl.BlockSpec((N // 10, D), lambda i: (i, 0)),
        out_shape=jax.ShapeDtypeStruct((N, D), jnp.float32),
    )(part)
    return out
